# baseline (speedup 1.0000x reference)
import functools

import jax
import jax.numpy as jnp
from jax import lax
from jax.experimental import pallas as pl
from jax.experimental.pallas import tpu as pltpu
from jax.experimental.pallas import tpu_sc as plsc

HIDDEN = 128
NUM_WORKERS = 32
PAIR = 2


def kernel(input_ids, weight):
    B, S = input_ids.shape
    bpw = B // NUM_WORKERS
    nsteps = S // PAIR
    idx_t = input_ids.astype(jnp.int32).T

    mesh = plsc.VectorSubcoreMesh(core_axis_name="c", subcore_axis_name="s")

    @functools.partial(
        pl.kernel,
        mesh=mesh,
        out_type=jax.ShapeDtypeStruct((S, B, HIDDEN), jnp.float32),
        compiler_params=pltpu.CompilerParams(use_tc_tiling_on_sc=True),
        scratch_types=[
            pltpu.VMEM((S, bpw), jnp.int32),
            pltpu.VMEM((2, PAIR, bpw, HIDDEN), jnp.float32),
            [pltpu.SemaphoreType.DMA] * 2,
            [pltpu.SemaphoreType.DMA] * 2,
        ],
    )
    def emb(table_hbm, idx_hbm, out_hbm, idx_v, rows_v, sem_in, sem_out):
        wid = lax.axis_index("s") * 2 + lax.axis_index("c")
        base = wid * bpw

        pltpu.sync_copy(idx_hbm.at[:, pl.ds(base, bpw)], idx_v)

        def start_gather(st, b):
            for i in range(PAIR):
                pltpu.make_async_copy(
                    table_hbm.at[idx_v.at[st * PAIR + i]],
                    rows_v.at[b, i], sem_in[b]).start()

        def wait_gather(b):
            for i in range(PAIR):
                pltpu.make_async_copy(
                    table_hbm.at[idx_v.at[0]],
                    rows_v.at[b, i], sem_in[b]).wait()

        def start_write(st, b):
            pltpu.make_async_copy(
                rows_v.at[b],
                out_hbm.at[pl.ds(st * PAIR, PAIR), pl.ds(base, bpw)],
                sem_out[b]).start()

        def wait_write(b):
            pltpu.make_async_copy(
                rows_v.at[b],
                out_hbm.at[pl.ds(0, PAIR), pl.ds(base, bpw)],
                sem_out[b]).wait()

        start_gather(0, 0)
        start_gather(1, 1)
        wait_gather(0)
        start_write(0, 0)

        def step(st, b):
            nb = 1 - b
            wait_write(nb)
            start_gather(st + 1, nb)
            wait_gather(b)
            start_write(st, b)

        def body(k, carry):
            step(2 * k + 1, 1)
            step(2 * k + 2, 0)
            return carry

        lax.fori_loop(0, (nsteps - 3) // 2, body, 0)

        step(nsteps - 2, 1)
        wait_gather(0)
        start_write(nsteps - 1, 0)
        wait_write(1)
        wait_write(0)

    out = emb(weight, idx_t)
    return out.transpose(1, 0, 2)


# speedup vs baseline: 1.0038x; 1.0038x over previous
import functools

import jax
import jax.numpy as jnp
from jax import lax
from jax.experimental import pallas as pl
from jax.experimental.pallas import tpu as pltpu
from jax.experimental.pallas import tpu_sc as plsc

HIDDEN = 128
NUM_WORKERS = 32
NBUF = 4


def kernel(input_ids, weight):
    B, S = input_ids.shape
    bpw = B // NUM_WORKERS
    idx_t = input_ids.astype(jnp.int32).T

    mesh = plsc.VectorSubcoreMesh(core_axis_name="c", subcore_axis_name="s")

    @functools.partial(
        pl.kernel,
        mesh=mesh,
        out_type=jax.ShapeDtypeStruct((S, B, HIDDEN), jnp.float32),
        compiler_params=pltpu.CompilerParams(use_tc_tiling_on_sc=True),
        scratch_types=[
            pltpu.VMEM((S, bpw), jnp.int32),
            pltpu.VMEM((NBUF, bpw, HIDDEN), jnp.float32),
            [pltpu.SemaphoreType.DMA] * NBUF,
            [pltpu.SemaphoreType.DMA] * NBUF,
            pltpu.SemaphoreType.DMA,
        ],
    )
    def emb(table_hbm, idx_hbm, out_hbm, idx_v, rows_v, sem_in, sem_out,
            sem_idx):
        wid = lax.axis_index("s") * 2 + lax.axis_index("c")
        base = wid * bpw

        head = 8
        pltpu.sync_copy(idx_hbm.at[pl.ds(0, head), pl.ds(base, bpw)],
                        idx_v.at[pl.ds(0, head)])
        rest = pltpu.make_async_copy(
            idx_hbm.at[pl.ds(head, S - head), pl.ds(base, bpw)],
            idx_v.at[pl.ds(head, S - head)], sem_idx)
        rest.start()

        def start_gather(s, b):
            pltpu.make_async_copy(
                table_hbm.at[idx_v.at[s]], rows_v.at[b], sem_in[b]).start()

        def wait_gather(b):
            pltpu.make_async_copy(
                table_hbm.at[idx_v.at[0]], rows_v.at[b], sem_in[b]).wait()

        def start_write(s, b):
            pltpu.make_async_copy(
                rows_v.at[b], out_hbm.at[s, pl.ds(base, bpw)],
                sem_out[b]).start()

        def wait_write(b):
            pltpu.make_async_copy(
                rows_v.at[b], out_hbm.at[0, pl.ds(base, bpw)],
                sem_out[b]).wait()

        start_gather(0, 0)
        start_gather(1, 1)
        start_gather(2, 2)
        wait_gather(0)
        start_write(0, 0)
        start_gather(3, 3)
        wait_gather(1)
        start_write(1, 1)
        rest.wait()

        def step(s, b):
            nb = (b + 2) % NBUF
            wait_write(nb)
            start_gather(s + 2, nb)
            wait_gather(b)
            start_write(s, b)

        def body(k, carry):
            for off in range(NBUF):
                s = NBUF * k + 2 + off
                step(s, (2 + off) % NBUF)
            return carry

        lax.fori_loop(0, (S - 6) // NBUF, body, 0)

        step(S - 4, (S - 4) % NBUF)
        step(S - 3, (S - 3) % NBUF)

        wait_gather((S - 2) % NBUF)
        start_write(S - 2, (S - 2) % NBUF)
        wait_gather((S - 1) % NBUF)
        start_write(S - 1, (S - 1) % NBUF)
        for b in range(NBUF):
            wait_write(b)

    out = emb(weight, idx_t)
    return out.transpose(1, 0, 2)
